# Initial kernel scaffold; baseline (speedup 1.0000x reference)
import jax
import jax.numpy as jnp
from jax.experimental import pallas as pl


def kernel(t_normalized, raw_betas):
    raise NotImplementedError("write your pallas kernel here")


# speedup vs baseline: 4.3796x; 4.3796x over previous
import functools

import jax
import jax.numpy as jnp
from jax import lax
from jax.experimental import pallas as pl
from jax.experimental.pallas import tpu as pltpu
from jax.experimental.pallas import tpu_sc as plsc

_N_TIMESTEPS = 1000
_BETA_MIN = 0.0001
_BETA_MAX = 0.02
_BATCH = 16384
_NC = 2
_NS = 16
_L = 16
_NW = _NC * _NS
_CHUNK = _BATCH // _NW
_STEPS = _CHUNK // _L


def _body(t_hbm, raw_hbm, out_hbm, tab_v, t_v, out_v):
    wid = lax.axis_index("s") * _NC + lax.axis_index("c")
    base = wid * _CHUNK
    pltpu.sync_copy(raw_hbm, tab_v)
    pltpu.sync_copy(t_hbm.at[pl.ds(base, _CHUNK)], t_v)
    for i in range(_STEPS):
        t16 = t_v[pl.ds(i * _L, _L)]
        idx16 = (t16 * float(_N_TIMESTEPS - 1)).astype(jnp.int32)
        g16 = plsc.load_gather(tab_v, [idx16])
        s16 = 1.0 / (1.0 + jnp.exp(-g16))
        out_v[pl.ds(i * _L, _L)] = s16 * (_BETA_MAX - _BETA_MIN) + _BETA_MIN
    pltpu.sync_copy(out_v, out_hbm.at[pl.ds(base, _CHUNK)])


@functools.partial(
    pl.kernel,
    out_type=jax.ShapeDtypeStruct((_BATCH,), jnp.float32),
    mesh=plsc.VectorSubcoreMesh(core_axis_name="c", subcore_axis_name="s"),
    compiler_params=pltpu.CompilerParams(needs_layout_passes=False),
    scratch_types=[
        pltpu.VMEM((_N_TIMESTEPS,), jnp.float32),
        pltpu.VMEM((_CHUNK,), jnp.float32),
        pltpu.VMEM((_CHUNK,), jnp.float32),
    ],
)
def _sc_noise_schedule(t_hbm, raw_hbm, out_hbm, tab_v, t_v, out_v):
    _body(t_hbm, raw_hbm, out_hbm, tab_v, t_v, out_v)


def kernel(t_normalized, raw_betas):
    return _sc_noise_schedule(t_normalized, raw_betas)


# speedup vs baseline: 4.4887x; 1.0249x over previous
import functools

import jax
import jax.numpy as jnp
from jax import lax
from jax.experimental import pallas as pl
from jax.experimental.pallas import tpu as pltpu
from jax.experimental.pallas import tpu_sc as plsc

_N_TIMESTEPS = 1000
_BETA_MIN = 0.0001
_BETA_MAX = 0.02
_BATCH = 16384
_NC = 2
_NS = 16
_L = 16
_NW = _NC * _NS
_CHUNK = _BATCH // _NW
_STEPS = _CHUNK // _L


def _body(t_hbm, raw_hbm, out_hbm, tab_v, t_v, out_v, sem_a, sem_b):
    wid = lax.axis_index("s") * _NC + lax.axis_index("c")
    base = wid * _CHUNK
    cp_tab = pltpu.async_copy(raw_hbm, tab_v, sem_a)
    cp_t = pltpu.async_copy(t_hbm.at[pl.ds(base, _CHUNK)], t_v, sem_b)
    cp_tab.wait()
    cp_t.wait()
    for i in range(_STEPS):
        t16 = t_v[pl.ds(i * _L, _L)]
        idx16 = (t16 * float(_N_TIMESTEPS - 1)).astype(jnp.int32)
        g16 = plsc.load_gather(tab_v, [idx16])
        s16 = 1.0 / (1.0 + jnp.exp(-g16))
        out_v[pl.ds(i * _L, _L)] = s16 * (_BETA_MAX - _BETA_MIN) + _BETA_MIN
    pltpu.sync_copy(out_v, out_hbm.at[pl.ds(base, _CHUNK)])


@functools.partial(
    pl.kernel,
    out_type=jax.ShapeDtypeStruct((_BATCH,), jnp.float32),
    mesh=plsc.VectorSubcoreMesh(core_axis_name="c", subcore_axis_name="s"),
    compiler_params=pltpu.CompilerParams(
        needs_layout_passes=False,
        disable_bounds_checks=True,
    ),
    scratch_types=[
        pltpu.VMEM((_N_TIMESTEPS,), jnp.float32),
        pltpu.VMEM((_CHUNK,), jnp.float32),
        pltpu.VMEM((_CHUNK,), jnp.float32),
        pltpu.SemaphoreType.DMA,
        pltpu.SemaphoreType.DMA,
    ],
)
def _sc_noise_schedule(t_hbm, raw_hbm, out_hbm, tab_v, t_v, out_v, sem_a, sem_b):
    _body(t_hbm, raw_hbm, out_hbm, tab_v, t_v, out_v, sem_a, sem_b)


def kernel(t_normalized, raw_betas):
    return _sc_noise_schedule(t_normalized, raw_betas)


# speedup vs baseline: 4.5060x; 1.0039x over previous
import functools

import jax
import jax.numpy as jnp
from jax import lax
from jax.experimental import pallas as pl
from jax.experimental.pallas import tpu as pltpu
from jax.experimental.pallas import tpu_sc as plsc

_N_TIMESTEPS = 1000
_BETA_MIN = 0.0001
_BETA_MAX = 0.02
_BATCH = 16384
_NC = 2
_NS = 16
_L = 16
_NW = _NC * _NS
_CHUNK = _BATCH // _NW
_STEPS = _CHUNK // _L


def _body(t_hbm, raw_hbm, out_hbm, tab_v, t_v, out_v, sem_a, sem_b):
    wid = lax.axis_index("s") * _NC + lax.axis_index("c")
    base = wid * _CHUNK
    cp_tab = pltpu.async_copy(raw_hbm, tab_v, sem_a)
    cp_t = pltpu.async_copy(t_hbm.at[pl.ds(base, _CHUNK)], t_v, sem_b)
    cp_tab.wait()
    cp_t.wait()
    for i in range(_STEPS):
        t16 = t_v[pl.ds(i * _L, _L)]
        idx16 = (t16 * float(_N_TIMESTEPS - 1)).astype(jnp.int32)
        g16 = plsc.load_gather(tab_v, [idx16])
        s16 = 1.0 / (1.0 + jnp.exp(-g16))
        out_v[pl.ds(i * _L, _L)] = s16 * (_BETA_MAX - _BETA_MIN) + _BETA_MIN
    pltpu.sync_copy(out_v, out_hbm.at[pl.ds(base, _CHUNK)])


@functools.partial(
    pl.kernel,
    out_type=jax.ShapeDtypeStruct((_BATCH,), jnp.float32),
    mesh=plsc.VectorSubcoreMesh(core_axis_name="c", subcore_axis_name="s"),
    compiler_params=pltpu.CompilerParams(
        needs_layout_passes=False,
        disable_bounds_checks=True,
        disable_semaphore_checks=True,
        skip_device_barrier=True,
    ),
    scratch_types=[
        pltpu.VMEM((_N_TIMESTEPS,), jnp.float32),
        pltpu.VMEM((_CHUNK,), jnp.float32),
        pltpu.VMEM((_CHUNK,), jnp.float32),
        pltpu.SemaphoreType.DMA,
        pltpu.SemaphoreType.DMA,
    ],
)
def _sc_noise_schedule(t_hbm, raw_hbm, out_hbm, tab_v, t_v, out_v, sem_a, sem_b):
    _body(t_hbm, raw_hbm, out_hbm, tab_v, t_v, out_v, sem_a, sem_b)


def kernel(t_normalized, raw_betas):
    return _sc_noise_schedule(t_normalized, raw_betas)
